# Initial kernel scaffold; baseline (speedup 1.0000x reference)
import jax
import jax.numpy as jnp
from jax.experimental import pallas as pl


def kernel(x, g, b_ln, W_qkv, W_out, b_out):
    raise NotImplementedError("write your pallas kernel here")


# speedup vs baseline: 1.6275x; 1.6275x over previous
import jax
import jax.numpy as jnp
from jax.experimental import pallas as pl

HEADS = 8
DIM_HEAD = 32
INNER = HEADS * DIM_HEAD
EPS = 1e-5


def _ln_qkv_body(x_ref, g_ref, b_ref, w_ref, o_ref):
    xb = x_ref[0]
    mean = jnp.mean(xb, axis=0, keepdims=True)
    xc = xb - mean
    var = jnp.mean(xc * xc, axis=0, keepdims=True)
    xn = xc * jax.lax.rsqrt(var + EPS) * g_ref[...] + b_ref[...]
    w = w_ref[...].astype(jnp.bfloat16)
    o_ref[0] = jnp.dot(w, xn.astype(jnp.bfloat16),
                       preferred_element_type=jnp.float32)


def _attn_body(q_ref, kt_ref, v_ref, o_ref):
    q = q_ref[0]
    kt = kt_ref[0]
    v = v_ref[0]
    qn = q / jnp.maximum(
        jnp.sqrt(jnp.sum(q * q, axis=-1, keepdims=True)), 1e-12)
    ktn = kt / jnp.maximum(
        jnp.sqrt(jnp.sum(kt * kt, axis=0, keepdims=True)), 1e-12)
    s = jnp.dot(qn.astype(jnp.bfloat16), ktn.astype(jnp.bfloat16),
                preferred_element_type=jnp.float32)
    e = jnp.exp(s)
    denom = jnp.sum(e, axis=-1, keepdims=True)
    o = jnp.dot(e.astype(jnp.bfloat16), v.astype(jnp.bfloat16),
                preferred_element_type=jnp.float32)
    o_ref[0] = o / denom


def _out_body(y_ref, w_ref, b_ref, o_ref):
    y = y_ref[0]
    w = w_ref[...]
    o_ref[0] = jnp.dot(w.astype(jnp.bfloat16), y.astype(jnp.bfloat16),
                       preferred_element_type=jnp.float32) + b_ref[...]


def kernel(x, g, b_ln, W_qkv, W_out, b_out):
    b, c, h, w = x.shape
    P = h * w
    BH = b * HEADS
    x2 = x.reshape(b, c, P)
    g2 = g.reshape(c, 1)
    b2 = b_ln.reshape(c, 1)

    qkv = pl.pallas_call(
        _ln_qkv_body,
        grid=(b,),
        in_specs=[
            pl.BlockSpec((1, c, P), lambda i: (i, 0, 0)),
            pl.BlockSpec((c, 1), lambda i: (0, 0)),
            pl.BlockSpec((c, 1), lambda i: (0, 0)),
            pl.BlockSpec((3 * INNER, c), lambda i: (0, 0)),
        ],
        out_specs=pl.BlockSpec((1, 3 * INNER, P), lambda i: (i, 0, 0)),
        out_shape=jax.ShapeDtypeStruct((b, 3 * INNER, P), jnp.float32),
    )(x2, g2, b2, W_qkv)

    q_tok = qkv[:, :INNER].reshape(BH, DIM_HEAD, P).reshape(BH, P, DIM_HEAD)
    k_tok = qkv[:, INNER:2 * INNER].reshape(BH, DIM_HEAD, P).reshape(
        BH, P, DIM_HEAD)
    kt = k_tok.transpose(0, 2, 1)
    v_tok = qkv[:, 2 * INNER:].reshape(BH, DIM_HEAD, P).reshape(
        BH, P, DIM_HEAD)

    o_tok = pl.pallas_call(
        _attn_body,
        grid=(BH,),
        in_specs=[
            pl.BlockSpec((1, P, DIM_HEAD), lambda i: (i, 0, 0)),
            pl.BlockSpec((1, DIM_HEAD, P), lambda i: (i, 0, 0)),
            pl.BlockSpec((1, P, DIM_HEAD), lambda i: (i, 0, 0)),
        ],
        out_specs=pl.BlockSpec((1, P, DIM_HEAD), lambda i: (i, 0, 0)),
        out_shape=jax.ShapeDtypeStruct((BH, P, DIM_HEAD), jnp.float32),
    )(q_tok, kt, v_tok)

    y = o_tok.reshape(b, HEADS, DIM_HEAD, h, w).transpose(0, 1, 3, 4, 2)
    y2 = y.reshape(b, INNER, P)

    out = pl.pallas_call(
        _out_body,
        grid=(b,),
        in_specs=[
            pl.BlockSpec((1, INNER, P), lambda i: (i, 0, 0)),
            pl.BlockSpec((c, INNER), lambda i: (0, 0)),
            pl.BlockSpec((c, 1), lambda i: (0, 0)),
        ],
        out_specs=pl.BlockSpec((1, c, P), lambda i: (i, 0, 0)),
        out_shape=jax.ShapeDtypeStruct((b, c, P), jnp.float32),
    )(y2, W_out, b_out.reshape(c, 1))

    return out.reshape(b, c, h, w)


# speedup vs baseline: 2.1022x; 1.2917x over previous
import jax
import jax.numpy as jnp
from jax.experimental import pallas as pl

HEADS = 8
DIM_HEAD = 32
INNER = HEADS * DIM_HEAD
EPS = 1e-5
D = DIM_HEAD


def _to_tok(a):
    b3 = a.T.reshape(D, D, D)
    b3 = b3.transpose(0, 2, 1)
    b3 = b3.transpose(1, 0, 2)
    return b3.reshape(D * D, D)


def _from_tok(o):
    o3 = o.reshape(D, D, D)
    y3 = o3.transpose(0, 2, 1)
    y3 = y3.transpose(1, 0, 2)
    return y3.reshape(D * D, D).T


def _ln_qkv_body(x_ref, g_ref, b_ref, w_ref, o_ref):
    xb = x_ref[0]
    mean = jnp.mean(xb, axis=0, keepdims=True)
    xc = xb - mean
    var = jnp.mean(xc * xc, axis=0, keepdims=True)
    xn = xc * jax.lax.rsqrt(var + EPS) * g_ref[...] + b_ref[...]
    w = w_ref[...].astype(jnp.bfloat16)
    o_ref[0] = jnp.dot(w, xn.astype(jnp.bfloat16),
                       preferred_element_type=jnp.float32)


def _attn_body(q_ref, k_ref, v_ref, o_ref):
    q = _to_tok(q_ref[0])
    k = _to_tok(k_ref[0])
    v = _to_tok(v_ref[0])
    qn = q / jnp.maximum(
        jnp.sqrt(jnp.sum(q * q, axis=-1, keepdims=True)), 1e-12)
    kn = k / jnp.maximum(
        jnp.sqrt(jnp.sum(k * k, axis=-1, keepdims=True)), 1e-12)
    s = jax.lax.dot_general(
        qn.astype(jnp.bfloat16), kn.astype(jnp.bfloat16),
        (((1,), (1,)), ((), ())),
        preferred_element_type=jnp.float32)
    e = jnp.exp(s)
    denom = jnp.sum(e, axis=-1, keepdims=True)
    o = jnp.dot(e.astype(jnp.bfloat16), v.astype(jnp.bfloat16),
                preferred_element_type=jnp.float32)
    o_ref[0] = _from_tok(o / denom)


def _out_body(y_ref, w_ref, b_ref, o_ref):
    y = y_ref[0]
    w = w_ref[...]
    o_ref[0] = jnp.dot(w.astype(jnp.bfloat16), y.astype(jnp.bfloat16),
                       preferred_element_type=jnp.float32) + b_ref[...]


def kernel(x, g, b_ln, W_qkv, W_out, b_out):
    b, c, h, w = x.shape
    P = h * w
    x2 = x.reshape(b, c, P)
    g2 = g.reshape(c, 1)
    b2 = b_ln.reshape(c, 1)

    qkv = pl.pallas_call(
        _ln_qkv_body,
        grid=(b,),
        in_specs=[
            pl.BlockSpec((1, c, P), lambda i: (i, 0, 0)),
            pl.BlockSpec((c, 1), lambda i: (0, 0)),
            pl.BlockSpec((c, 1), lambda i: (0, 0)),
            pl.BlockSpec((3 * INNER, c), lambda i: (0, 0)),
        ],
        out_specs=pl.BlockSpec((1, 3 * INNER, P), lambda i: (i, 0, 0)),
        out_shape=jax.ShapeDtypeStruct((b, 3 * INNER, P), jnp.float32),
    )(x2, g2, b2, W_qkv)

    y2 = pl.pallas_call(
        _attn_body,
        grid=(b * HEADS,),
        in_specs=[
            pl.BlockSpec((1, DIM_HEAD, P),
                         lambda i: (i // HEADS, i % HEADS, 0)),
            pl.BlockSpec((1, DIM_HEAD, P),
                         lambda i: (i // HEADS, HEADS + i % HEADS, 0)),
            pl.BlockSpec((1, DIM_HEAD, P),
                         lambda i: (i // HEADS, 2 * HEADS + i % HEADS, 0)),
        ],
        out_specs=pl.BlockSpec((1, DIM_HEAD, P),
                               lambda i: (i // HEADS, i % HEADS, 0)),
        out_shape=jax.ShapeDtypeStruct((b, INNER, P), jnp.float32),
    )(qkv, qkv, qkv)

    out = pl.pallas_call(
        _out_body,
        grid=(b,),
        in_specs=[
            pl.BlockSpec((1, INNER, P), lambda i: (i, 0, 0)),
            pl.BlockSpec((c, INNER), lambda i: (0, 0)),
            pl.BlockSpec((c, 1), lambda i: (0, 0)),
        ],
        out_specs=pl.BlockSpec((1, c, P), lambda i: (i, 0, 0)),
        out_shape=jax.ShapeDtypeStruct((b, c, P), jnp.float32),
    )(y2, W_out, b_out.reshape(c, 1))

    return out.reshape(b, c, h, w)


# speedup vs baseline: 2.8223x; 1.3426x over previous
import jax
import jax.numpy as jnp
from jax.experimental import pallas as pl

HEADS = 8
DIM_HEAD = 32
INNER = HEADS * DIM_HEAD
EPS = 1e-5
D = DIM_HEAD


def _from_tok(o):
    o3 = o.reshape(D, D, D)
    y3 = o3.transpose(0, 2, 1)
    y3 = y3.transpose(1, 0, 2)
    return y3.reshape(D * D, D).T


def _fused_body(x_ref, g_ref, b_ref, wqkv_ref, wout_ref, bout_ref, o_ref):
    C = x_ref.shape[1]
    G = 3 * HEADS
    xb = x_ref[0]
    mean = jnp.mean(xb, axis=0, keepdims=True)
    xc = xb - mean
    var = jnp.mean(xc * xc, axis=0, keepdims=True)
    xn = xc * jax.lax.rsqrt(var + EPS) * g_ref[...] + b_ref[...]
    qkv = jnp.dot(wqkv_ref[...].astype(jnp.bfloat16),
                  xn.astype(jnp.bfloat16),
                  preferred_element_type=jnp.float32)

    t = qkv.T.reshape(D, D, G * D)
    t = t.transpose(0, 2, 1)
    t = t.transpose(1, 0, 2)
    t = t.reshape(G, D, D, D)
    t = t.reshape(G, D * D, D)

    ys = []
    for g in range(HEADS):
        q = t[g]
        k = t[HEADS + g]
        v = t[2 * HEADS + g]
        qn = q / jnp.maximum(
            jnp.sqrt(jnp.sum(q * q, axis=-1, keepdims=True)), 1e-12)
        kn = k / jnp.maximum(
            jnp.sqrt(jnp.sum(k * k, axis=-1, keepdims=True)), 1e-12)
        s = jax.lax.dot_general(
            qn.astype(jnp.bfloat16), kn.astype(jnp.bfloat16),
            (((1,), (1,)), ((), ())),
            preferred_element_type=jnp.float32)
        e = jnp.exp(s)
        denom = jnp.sum(e, axis=-1, keepdims=True)
        o = jnp.dot(e.astype(jnp.bfloat16), v.astype(jnp.bfloat16),
                    preferred_element_type=jnp.float32)
        ys.append(_from_tok(o / denom))
    y = jnp.concatenate(ys, axis=0)

    o_ref[0] = jnp.dot(wout_ref[...].astype(jnp.bfloat16),
                       y.astype(jnp.bfloat16),
                       preferred_element_type=jnp.float32) + bout_ref[...]


def kernel(x, g, b_ln, W_qkv, W_out, b_out):
    b, c, h, w = x.shape
    P = h * w
    x2 = x.reshape(b, c, P)

    out = pl.pallas_call(
        _fused_body,
        grid=(b,),
        in_specs=[
            pl.BlockSpec((1, c, P), lambda i: (i, 0, 0)),
            pl.BlockSpec((c, 1), lambda i: (0, 0)),
            pl.BlockSpec((c, 1), lambda i: (0, 0)),
            pl.BlockSpec((3 * INNER, c), lambda i: (0, 0)),
            pl.BlockSpec((c, INNER), lambda i: (0, 0)),
            pl.BlockSpec((c, 1), lambda i: (0, 0)),
        ],
        out_specs=pl.BlockSpec((1, c, P), lambda i: (i, 0, 0)),
        out_shape=jax.ShapeDtypeStruct((b, c, P), jnp.float32),
    )(x2, g.reshape(c, 1), b_ln.reshape(c, 1), W_qkv, W_out,
      b_out.reshape(c, 1))

    return out.reshape(b, c, h, w)


# speedup vs baseline: 3.1554x; 1.1180x over previous
import jax
import jax.numpy as jnp
from jax.experimental import pallas as pl

HEADS = 8
DIM_HEAD = 32
INNER = HEADS * DIM_HEAD
EPS = 1e-5
D = DIM_HEAD


def _from_tok(o):
    o3 = o.reshape(D, D, D)
    y3 = o3.transpose(0, 2, 1)
    y3 = y3.transpose(1, 0, 2)
    return y3.reshape(D * D, D).T


def _fused_body(x_ref, g_ref, b_ref, wqkv_ref, wout_ref, bout_ref, o_ref):
    C = x_ref.shape[1]
    G = 3 * HEADS
    xb = x_ref[0]
    mean = jnp.mean(xb, axis=0, keepdims=True)
    xc = xb - mean
    var = jnp.mean(xc * xc, axis=0, keepdims=True)
    xn = xc * jax.lax.rsqrt(var + EPS) * g_ref[...] + b_ref[...]
    qkv = jnp.dot(wqkv_ref[...].astype(jnp.bfloat16),
                  xn.astype(jnp.bfloat16),
                  preferred_element_type=jnp.float32)

    thwc = qkv.T.reshape(D, D, G * D)
    nsq = jnp.sum(thwc[:, :, :2 * INNER] ** 2, axis=1, keepdims=True)
    rn = 1.0 / jnp.maximum(jnp.sqrt(nsq), 1e-12)
    rn_full = jnp.concatenate(
        [rn, jnp.ones((D, 1, INNER), jnp.float32)], axis=2)
    thwc = thwc * rn_full
    t = thwc.transpose(0, 2, 1)
    t = t.transpose(1, 0, 2)
    t = t.reshape(G, D, D, D)
    t = t.reshape(G, D * D, D)

    ones_col = jnp.ones((D * D, 1), jnp.bfloat16)
    ys = []
    for g in range(HEADS):
        qn = t[g]
        kn = t[HEADS + g]
        v = t[2 * HEADS + g]
        s = jax.lax.dot_general(
            qn.astype(jnp.bfloat16), kn.astype(jnp.bfloat16),
            (((1,), (1,)), ((), ())),
            preferred_element_type=jnp.float32)
        e = jnp.exp(s).astype(jnp.bfloat16)
        vaug = jnp.concatenate([v.astype(jnp.bfloat16), ones_col], axis=1)
        of = jnp.dot(e, vaug, preferred_element_type=jnp.float32)
        o = of[:, :D] / of[:, D:]
        ys.append(_from_tok(o))
    y = jnp.concatenate(ys, axis=0)

    o_ref[0] = jnp.dot(wout_ref[...].astype(jnp.bfloat16),
                       y.astype(jnp.bfloat16),
                       preferred_element_type=jnp.float32) + bout_ref[...]


def kernel(x, g, b_ln, W_qkv, W_out, b_out):
    b, c, h, w = x.shape
    P = h * w
    x2 = x.reshape(b, c, P)

    out = pl.pallas_call(
        _fused_body,
        grid=(b,),
        in_specs=[
            pl.BlockSpec((1, c, P), lambda i: (i, 0, 0)),
            pl.BlockSpec((c, 1), lambda i: (0, 0)),
            pl.BlockSpec((c, 1), lambda i: (0, 0)),
            pl.BlockSpec((3 * INNER, c), lambda i: (0, 0)),
            pl.BlockSpec((c, INNER), lambda i: (0, 0)),
            pl.BlockSpec((c, 1), lambda i: (0, 0)),
        ],
        out_specs=pl.BlockSpec((1, c, P), lambda i: (i, 0, 0)),
        out_shape=jax.ShapeDtypeStruct((b, c, P), jnp.float32),
    )(x2, g.reshape(c, 1), b_ln.reshape(c, 1), W_qkv, W_out,
      b_out.reshape(c, 1))

    return out.reshape(b, c, h, w)


# speedup vs baseline: 3.3167x; 1.0511x over previous
import jax
import jax.numpy as jnp
from jax.experimental import pallas as pl

HEADS = 8
DIM_HEAD = 32
INNER = HEADS * DIM_HEAD
EPS = 1e-5
D = DIM_HEAD


def _from_tok(o):
    o3 = o.reshape(D, D, D)
    y3 = o3.transpose(0, 2, 1)
    y3 = y3.transpose(1, 0, 2)
    return y3.reshape(D * D, D).T


def _fused_body(x_ref, g_ref, b_ref, wqkv_ref, wout_ref, bout_ref, o_ref):
    C = x_ref.shape[1]
    G = 3 * HEADS
    xb = x_ref[0]
    mean = jnp.mean(xb, axis=0, keepdims=True)
    xc = xb - mean
    var = jnp.mean(xc * xc, axis=0, keepdims=True)
    xn = xc * jax.lax.rsqrt(var + EPS) * g_ref[...] + b_ref[...]
    qkv = jnp.dot(wqkv_ref[...].astype(jnp.bfloat16),
                  xn.astype(jnp.bfloat16),
                  preferred_element_type=jnp.float32)

    thwc = qkv.T.reshape(D, D, G * D)
    nsq = jnp.sum(thwc[:, :, :2 * INNER] ** 2, axis=1, keepdims=True)
    rn = 1.0 / jnp.maximum(jnp.sqrt(nsq), 1e-12)
    rn_full = jnp.concatenate(
        [rn, jnp.ones((D, 1, INNER), jnp.float32)], axis=2)
    thwc = (thwc * rn_full).astype(jnp.bfloat16)
    t = thwc.transpose(0, 2, 1)
    t = t.transpose(1, 0, 2)
    t = t.reshape(G, D, D, D)
    t = t.reshape(G, D * D, D)

    ones_col = jnp.ones((D * D, 1), jnp.bfloat16)
    ys = []
    for g in range(HEADS):
        qn = t[g]
        kn = t[HEADS + g]
        v = t[2 * HEADS + g]
        s = jax.lax.dot_general(
            qn, kn, (((1,), (1,)), ((), ())),
            preferred_element_type=jnp.float32)
        e = jnp.exp(s).astype(jnp.bfloat16)
        vaug = jnp.concatenate([v, ones_col], axis=1)
        of = jnp.dot(e, vaug, preferred_element_type=jnp.float32)
        o = (of[:, :D] / of[:, D:]).astype(jnp.bfloat16)
        ys.append(_from_tok(o))
    y = jnp.concatenate(ys, axis=0)

    o_ref[0] = jnp.dot(wout_ref[...].astype(jnp.bfloat16), y,
                       preferred_element_type=jnp.float32) + bout_ref[...]


def kernel(x, g, b_ln, W_qkv, W_out, b_out):
    b, c, h, w = x.shape
    P = h * w
    x2 = x.reshape(b, c, P)

    out = pl.pallas_call(
        _fused_body,
        grid=(b,),
        in_specs=[
            pl.BlockSpec((1, c, P), lambda i: (i, 0, 0)),
            pl.BlockSpec((c, 1), lambda i: (0, 0)),
            pl.BlockSpec((c, 1), lambda i: (0, 0)),
            pl.BlockSpec((3 * INNER, c), lambda i: (0, 0)),
            pl.BlockSpec((c, INNER), lambda i: (0, 0)),
            pl.BlockSpec((c, 1), lambda i: (0, 0)),
        ],
        out_specs=pl.BlockSpec((1, c, P), lambda i: (i, 0, 0)),
        out_shape=jax.ShapeDtypeStruct((b, c, P), jnp.float32),
    )(x2, g.reshape(c, 1), b_ln.reshape(c, 1), W_qkv, W_out,
      b_out.reshape(c, 1))

    return out.reshape(b, c, h, w)


# speedup vs baseline: 3.3169x; 1.0001x over previous
import jax
import jax.numpy as jnp
from jax.experimental import pallas as pl
from jax.experimental.pallas import tpu as pltpu

HEADS = 8
DIM_HEAD = 32
INNER = HEADS * DIM_HEAD
EPS = 1e-5
D = DIM_HEAD


def _from_tok(o):
    o3 = o.reshape(D, D, D)
    y3 = o3.transpose(0, 2, 1)
    y3 = y3.transpose(1, 0, 2)
    return y3.reshape(D * D, D).T


def _fused_body(x_ref, g_ref, b_ref, wqkv_ref, wout_ref, bout_ref, o_ref):
    C = x_ref.shape[1]
    G = 3 * HEADS
    xb = x_ref[0]
    mean = jnp.mean(xb, axis=0, keepdims=True)
    xc = xb - mean
    var = jnp.mean(xc * xc, axis=0, keepdims=True)
    xn = xc * jax.lax.rsqrt(var + EPS) * g_ref[...] + b_ref[...]
    qkv = jnp.dot(wqkv_ref[...].astype(jnp.bfloat16),
                  xn.astype(jnp.bfloat16),
                  preferred_element_type=jnp.float32)

    thwc = qkv.T.reshape(D, D, G * D)
    nsq = jnp.sum(thwc[:, :, :2 * INNER] ** 2, axis=1, keepdims=True)
    rn = 1.0 / jnp.maximum(jnp.sqrt(nsq), 1e-12)
    rn_full = jnp.concatenate(
        [rn, jnp.ones((D, 1, INNER), jnp.float32)], axis=2)
    thwc = (thwc * rn_full).astype(jnp.bfloat16)
    t = thwc.transpose(0, 2, 1)
    t = t.transpose(1, 0, 2)
    t = t.reshape(G, D, D, D)
    t = t.reshape(G, D * D, D)

    ones_col = jnp.ones((D * D, 1), jnp.bfloat16)
    ys = []
    for g in range(HEADS):
        qn = t[g]
        kn = t[HEADS + g]
        v = t[2 * HEADS + g]
        s = jax.lax.dot_general(
            qn, kn, (((1,), (1,)), ((), ())),
            preferred_element_type=jnp.float32)
        e = jnp.exp(s).astype(jnp.bfloat16)
        vaug = jnp.concatenate([v, ones_col], axis=1)
        of = jnp.dot(e, vaug, preferred_element_type=jnp.float32)
        o = (of[:, :D] / of[:, D:]).astype(jnp.bfloat16)
        ys.append(_from_tok(o))
    y = jnp.concatenate(ys, axis=0)

    o_ref[0] = jnp.dot(wout_ref[...].astype(jnp.bfloat16), y,
                       preferred_element_type=jnp.float32) + bout_ref[...]


def kernel(x, g, b_ln, W_qkv, W_out, b_out):
    b, c, h, w = x.shape
    P = h * w
    x2 = x.reshape(b, c, P)

    out = pl.pallas_call(
        _fused_body,
        grid=(b,),
        in_specs=[
            pl.BlockSpec((1, c, P), lambda i: (i, 0, 0)),
            pl.BlockSpec((c, 1), lambda i: (0, 0)),
            pl.BlockSpec((c, 1), lambda i: (0, 0)),
            pl.BlockSpec((3 * INNER, c), lambda i: (0, 0)),
            pl.BlockSpec((c, INNER), lambda i: (0, 0)),
            pl.BlockSpec((c, 1), lambda i: (0, 0)),
        ],
        out_specs=pl.BlockSpec((1, c, P), lambda i: (i, 0, 0)),
        out_shape=jax.ShapeDtypeStruct((b, c, P), jnp.float32),
        compiler_params=pltpu.CompilerParams(
            dimension_semantics=("parallel",)),
    )(x2, g.reshape(c, 1), b_ln.reshape(c, 1), W_qkv, W_out,
      b_out.reshape(c, 1))

    return out.reshape(b, c, h, w)
